# baseline (speedup 1.0000x reference)
import functools

import jax
import jax.numpy as jnp
from jax import lax
from jax.experimental import pallas as pl
from jax.experimental.pallas import tpu as pltpu
from jax.experimental.pallas import tpu_sc as plsc

_F = 128
_OUTW = 2 * _F
_NC = 2
_NS = 16
_NW = _NC * _NS
_LANES = 16
_CHUNK = 64


@functools.lru_cache(maxsize=None)
def _make_kernel(P: int):
    assert P % _NW == 0
    ppw = P // _NW
    assert ppw % _CHUNK == 0
    nch = ppw // _CHUNK

    mesh = plsc.VectorSubcoreMesh(
        core_axis_name="c", subcore_axis_name="s",
        num_cores=_NC, num_subcores=_NS)

    @functools.partial(
        pl.kernel,
        out_type=jax.ShapeDtypeStruct((P * _OUTW,), jnp.float32),
        mesh=mesh,
        scratch_types=[
            pltpu.VMEM((ppw,), jnp.int32),
            pltpu.VMEM((ppw,), jnp.int32),
            pltpu.VMEM((_CHUNK, _F), jnp.float32),
            pltpu.VMEM((_CHUNK, _F), jnp.float32),
            pltpu.VMEM((_CHUNK * _OUTW,), jnp.float32),
            pltpu.SemaphoreType.DMA,
        ],
        compiler_params=pltpu.CompilerParams(needs_layout_passes=False),
    )
    def emb(x0_hbm, x1_hbm, col_hbm, row_hbm, out_hbm,
            idx0, idx1, buf_a, buf_b, buf_c, sem):
        wid = lax.axis_index("s") * _NC + lax.axis_index("c")
        base = wid * ppw
        pltpu.sync_copy(x0_hbm.at[pl.ds(base, ppw)], idx0)
        pltpu.sync_copy(x1_hbm.at[pl.ds(base, ppw)], idx1)
        ev = 2 * lax.iota(jnp.int32, _LANES)

        def do_chunk(ci, carry):
            off = ci * _CHUNK
            ga = pltpu.async_copy(
                col_hbm.at[idx0.at[pl.ds(off, _CHUNK)]], buf_a, sem)
            gb = pltpu.async_copy(
                row_hbm.at[idx1.at[pl.ds(off, _CHUNK)]], buf_b, sem)
            ga.wait()
            gb.wait()

            def do_point(p, c2):
                pb = p * _OUTW
                for j in range(_F // _LANES):
                    va = buf_a[p, pl.ds(j * _LANES, _LANES)]
                    plsc.store_scatter(buf_c, [pb + 2 * j * _LANES + ev], va)
                    vb = buf_b[p, pl.ds(j * _LANES, _LANES)]
                    plsc.store_scatter(buf_c, [pb + 2 * j * _LANES + 1 + ev], vb)
                return c2

            lax.fori_loop(0, _CHUNK, do_point, 0)
            pltpu.sync_copy(
                buf_c, out_hbm.at[pl.ds((base + off) * _OUTW, _CHUNK * _OUTW)])
            return carry

        lax.fori_loop(0, nch, do_chunk, 0)

    return emb


def kernel(x, col_embed, row_embed):
    b, n, _ = x.shape
    p = b * n
    x0 = x[..., 0].reshape(p)
    x1 = x[..., 1].reshape(p)
    out = _make_kernel(p)(x0, x1, col_embed, row_embed)
    return out.reshape(b, n, _F, 2)
